# baseline (speedup 1.0000x reference)
import functools

import jax
import jax.numpy as jnp
from jax.experimental import pallas as pl
from jax.experimental.pallas import tpu as pltpu

E = 8
K = 2
BLK = 256
DI_T = 1024
TB = 512
PB = TB * K


def _router_body(x_ref, rw_ref, rb_ref, e_ref, r_ref, w_ref, cnt_ref, carry):
    c = pl.program_id(0)

    @pl.when(c == 0)
    def _():
        carry[...] = jnp.zeros_like(carry)

    lg = jax.lax.dot_general(rw_ref[...].astype(jnp.bfloat16),
                             x_ref[...].astype(jnp.bfloat16),
                             (((1,), (1,)), ((), ())),
                             preferred_element_type=jnp.float32)
    lg = lg + rb_ref[...].reshape(E, 1)
    m = jnp.max(lg, axis=0, keepdims=True)
    q = jnp.exp(lg - m)
    p = q / jnp.sum(q, axis=0, keepdims=True)
    iota = jax.lax.broadcasted_iota(jnp.int32, (E, TB), 0)
    m1 = jnp.max(p, axis=0, keepdims=True)
    i1 = jnp.min(jnp.where(p == m1, iota, E), axis=0, keepdims=True)
    masked = jnp.where(iota == i1, -1.0, p)
    m2 = jnp.max(masked, axis=0, keepdims=True)
    i2 = jnp.min(jnp.where((masked == m2) & (iota != i1), iota, E),
                 axis=0, keepdims=True)
    s = m1 + m2
    w1 = m1 / s
    w2 = m2 / s

    oh = jnp.concatenate([(iota == i1), (iota == i2)],
                         axis=1).astype(jnp.float32)
    pr = jax.lax.broadcasted_iota(jnp.int32, (PB, PB), 0)
    pc = jax.lax.broadcasted_iota(jnp.int32, (PB, PB), 1)
    ltu = (pr < pc).astype(jnp.float32)
    cum = jax.lax.dot_general(oh, ltu, (((1,), (0,)), ((), ())),
                              preferred_element_type=jnp.float32,
                              precision=jax.lax.Precision.HIGHEST)
    rank = (jnp.sum(oh * cum, axis=0, keepdims=True)
            + jax.lax.dot_general(carry[...], oh, (((1,), (0,)), ((), ())),
                                  preferred_element_type=jnp.float32,
                                  precision=jax.lax.Precision.HIGHEST))

    evec = jnp.concatenate([i1, i2], axis=1)
    wvec = jnp.concatenate([w1, w2], axis=1)
    e_ref[...] = evec.reshape(PB)
    r_ref[...] = rank.astype(jnp.int32).reshape(PB)
    w_ref[...] = wvec.reshape(PB)
    carry[...] += jax.lax.dot_general(
        jnp.ones((1, PB), jnp.float32), oh, (((1,), (1,)), ((), ())),
        preferred_element_type=jnp.float32,
        precision=jax.lax.Precision.HIGHEST)
    cnt_ref[...] = carry[...].astype(jnp.int32).reshape(E)


def _router(flat_x, router_w, router_b):
    t, d = flat_x.shape
    nc = t // TB
    tk = t * K
    grid_spec = pltpu.PrefetchScalarGridSpec(
        num_scalar_prefetch=0,
        grid=(nc,),
        in_specs=[
            pl.BlockSpec((TB, d), lambda c: (c, 0)),
            pl.BlockSpec((E, d), lambda c: (0, 0)),
            pl.BlockSpec((1, E), lambda c: (0, 0)),
        ],
        out_specs=[
            pl.BlockSpec((PB,), lambda c: (c,)),
            pl.BlockSpec((PB,), lambda c: (c,)),
            pl.BlockSpec((PB,), lambda c: (c,)),
            pl.BlockSpec((E,), lambda c: (0,)),
        ],
        scratch_shapes=[pltpu.VMEM((1, E), jnp.float32)],
    )
    return pl.pallas_call(
        _router_body,
        grid_spec=grid_spec,
        out_shape=[
            jax.ShapeDtypeStruct((tk,), jnp.int32),
            jax.ShapeDtypeStruct((tk,), jnp.int32),
            jax.ShapeDtypeStruct((tk,), jnp.float32),
            jax.ShapeDtypeStruct((E,), jnp.int32),
        ],
    )(flat_x, router_w, router_b.reshape(1, E))


def _ffn_body(be_ref, xs_ref, gw_ref, uw_ref, dw_ref, out_ref):
    j = pl.program_id(1)
    x = xs_ref[...]
    g = jax.lax.dot_general(x, gw_ref[0], (((1,), (1,)), ((), ())),
                            preferred_element_type=jnp.float32)
    u = jax.lax.dot_general(x, uw_ref[0], (((1,), (1,)), ((), ())),
                            preferred_element_type=jnp.float32)
    h = g * jax.nn.sigmoid(g) * u
    o = jax.lax.dot_general(h, dw_ref[0], (((1,), (1,)), ((), ())),
                            preferred_element_type=jnp.float32)

    @pl.when(j == 0)
    def _():
        out_ref[...] = o

    @pl.when(j > 0)
    def _():
        out_ref[...] += o


def _grouped_ffn(xs, gate_w, up_w, down_w, block_expert):
    pt, d = xs.shape
    e, di, _ = gate_w.shape
    nb = pt // BLK
    it = di // DI_T
    grid_spec = pltpu.PrefetchScalarGridSpec(
        num_scalar_prefetch=1,
        grid=(nb, it),
        in_specs=[
            pl.BlockSpec((BLK, d), lambda i, j, be: (i, 0)),
            pl.BlockSpec((1, DI_T, d), lambda i, j, be: (be[i], j, 0)),
            pl.BlockSpec((1, DI_T, d), lambda i, j, be: (be[i], j, 0)),
            pl.BlockSpec((1, d, DI_T), lambda i, j, be: (be[i], 0, j)),
        ],
        out_specs=pl.BlockSpec((BLK, d), lambda i, j, be: (i, 0)),
    )
    return pl.pallas_call(
        _ffn_body,
        grid_spec=grid_spec,
        out_shape=jax.ShapeDtypeStruct((pt, d), jnp.float32),
    )(block_expert, xs, gate_w, up_w, down_w)


def kernel(x, router_w, router_b, gate_w, up_w, down_w):
    b, s, d = x.shape
    t = b * s
    tk = t * K
    pt = tk + E * BLK
    nb = pt // BLK

    flat_x = x.reshape(t, d)

    experts, ranks, wts, counts = _router(flat_x, router_w, router_b)

    padded = ((counts + BLK - 1) // BLK) * BLK
    pend = jnp.cumsum(padded)
    poff = (pend - padded).astype(jnp.int32)
    block_expert = jnp.minimum(
        jnp.searchsorted(pend, jnp.arange(nb, dtype=jnp.int32) * BLK,
                         side='right'),
        E - 1).astype(jnp.int32)

    dest = poff[experts] + ranks

    p_ar = jnp.arange(tk, dtype=jnp.int32)
    tok_of_pair = TB * (p_ar // PB) + (p_ar % TB)

    src = jnp.zeros((pt,), jnp.int32).at[dest].set(tok_of_pair)
    xs = flat_x[src]

    outs = _grouped_ffn(xs, gate_w, up_w, down_w, block_expert)

    t_ar = jnp.arange(t, dtype=jnp.int32)
    pos0 = PB * (t_ar // TB) + (t_ar % TB)
    pos1 = pos0 + TB
    y = (outs[dest[pos0]] * wts[pos0][:, None]
         + outs[dest[pos1]] * wts[pos1][:, None])
    return y.reshape(b, s, d)


# speedup vs baseline: 1.2605x; 1.2605x over previous
import functools

import jax
import jax.numpy as jnp
from jax import lax
from jax.experimental import pallas as pl
from jax.experimental.pallas import tpu as pltpu
from jax.experimental.pallas import tpu_sc as plsc

E = 8
K = 2
BLK = 256
DI_T = 1024
TB = 512
PB = TB * K

NC = 2
NS = 16
NW = NC * NS
L = 16


def _router_body(x_ref, rw_ref, rb_ref, e_ref, r_ref, w_ref, cnt_ref, carry):
    c = pl.program_id(0)

    @pl.when(c == 0)
    def _():
        carry[...] = jnp.zeros_like(carry)

    lg = jax.lax.dot_general(rw_ref[...].astype(jnp.bfloat16),
                             x_ref[...].astype(jnp.bfloat16),
                             (((1,), (1,)), ((), ())),
                             preferred_element_type=jnp.float32)
    lg = lg + rb_ref[...].reshape(E, 1)
    m = jnp.max(lg, axis=0, keepdims=True)
    q = jnp.exp(lg - m)
    p = q / jnp.sum(q, axis=0, keepdims=True)
    iota = jax.lax.broadcasted_iota(jnp.int32, (E, TB), 0)
    m1 = jnp.max(p, axis=0, keepdims=True)
    i1 = jnp.min(jnp.where(p == m1, iota, E), axis=0, keepdims=True)
    masked = jnp.where(iota == i1, -1.0, p)
    m2 = jnp.max(masked, axis=0, keepdims=True)
    i2 = jnp.min(jnp.where((masked == m2) & (iota != i1), iota, E),
                 axis=0, keepdims=True)
    s = m1 + m2
    w1 = m1 / s
    w2 = m2 / s

    oh = jnp.concatenate([(iota == i1), (iota == i2)],
                         axis=1).astype(jnp.float32)
    pr = jax.lax.broadcasted_iota(jnp.int32, (PB, PB), 0)
    pc = jax.lax.broadcasted_iota(jnp.int32, (PB, PB), 1)
    ltu = (pr < pc).astype(jnp.float32)
    cum = jax.lax.dot_general(oh, ltu, (((1,), (0,)), ((), ())),
                              preferred_element_type=jnp.float32,
                              precision=jax.lax.Precision.HIGHEST)
    rank = (jnp.sum(oh * cum, axis=0, keepdims=True)
            + jax.lax.dot_general(carry[...], oh, (((1,), (0,)), ((), ())),
                                  preferred_element_type=jnp.float32,
                                  precision=jax.lax.Precision.HIGHEST))

    evec = jnp.concatenate([i1, i2], axis=1)
    wvec = jnp.concatenate([w1, w2], axis=1)
    e_ref[...] = evec.reshape(PB)
    r_ref[...] = rank.astype(jnp.int32).reshape(PB)
    w_ref[...] = wvec.reshape(PB)
    carry[...] += jax.lax.dot_general(
        jnp.ones((1, PB), jnp.float32), oh, (((1,), (1,)), ((), ())),
        preferred_element_type=jnp.float32,
        precision=jax.lax.Precision.HIGHEST)
    cnt_ref[...] = carry[...].astype(jnp.int32).reshape(E)


def _router(flat_x, router_w, router_b):
    t, d = flat_x.shape
    nc = t // TB
    tk = t * K
    grid_spec = pltpu.PrefetchScalarGridSpec(
        num_scalar_prefetch=0,
        grid=(nc,),
        in_specs=[
            pl.BlockSpec((TB, d), lambda c: (c, 0)),
            pl.BlockSpec((E, d), lambda c: (0, 0)),
            pl.BlockSpec((1, E), lambda c: (0, 0)),
        ],
        out_specs=[
            pl.BlockSpec((PB,), lambda c: (c,)),
            pl.BlockSpec((PB,), lambda c: (c,)),
            pl.BlockSpec((PB,), lambda c: (c,)),
            pl.BlockSpec((E,), lambda c: (0,)),
        ],
        scratch_shapes=[pltpu.VMEM((1, E), jnp.float32)],
    )
    return pl.pallas_call(
        _router_body,
        grid_spec=grid_spec,
        out_shape=[
            jax.ShapeDtypeStruct((tk,), jnp.int32),
            jax.ShapeDtypeStruct((tk,), jnp.int32),
            jax.ShapeDtypeStruct((tk,), jnp.float32),
            jax.ShapeDtypeStruct((E,), jnp.int32),
        ],
    )(flat_x, router_w, router_b.reshape(1, E))


def _dispatch(flat_x, experts, ranks, poff, pt):
    t, d = flat_x.shape
    tk = t * K
    ppw = tk // NW
    nchunks = ppw // L

    mesh = plsc.VectorSubcoreMesh(core_axis_name="c", subcore_axis_name="s")

    @functools.partial(
        pl.kernel,
        mesh=mesh,
        compiler_params=pltpu.CompilerParams(needs_layout_passes=False),
        out_type=[
            jax.ShapeDtypeStruct((pt, d), jnp.float32),
            jax.ShapeDtypeStruct((tk,), jnp.int32),
        ],
        scratch_types=[
            pltpu.VMEM((ppw,), jnp.int32),
            pltpu.VMEM((ppw,), jnp.int32),
            pltpu.VMEM((L,), jnp.int32),
            pltpu.VMEM((ppw,), jnp.int32),
            pltpu.VMEM((L, 1024), jnp.float32),
            pltpu.SemaphoreType.DMA,
        ],
    )
    def k(x_hbm, e_hbm, r_hbm, poff_hbm, xs_hbm, dest_hbm,
          evm, rvm, pvm, dflat, rows, sem):
        wid = lax.axis_index("s") * NC + lax.axis_index("c")
        p0 = wid * ppw
        chunk = p0 // PB
        tbase = TB * chunk + (p0 % TB)

        pltpu.sync_copy(e_hbm.at[pl.ds(p0, ppw)], evm)
        pltpu.sync_copy(r_hbm.at[pl.ds(p0, ppw)], rvm)
        pltpu.sync_copy(poff_hbm, pvm.at[pl.ds(0, E)])

        for i in range(nchunks):
            ev = evm[pl.ds(i * L, L)]
            rv = rvm[pl.ds(i * L, L)]
            dflat[pl.ds(i * L, L)] = plsc.load_gather(pvm, [ev]) + rv

        pltpu.sync_copy(dflat, dest_hbm.at[pl.ds(p0, ppw)])

        def body(i, _):
            pltpu.sync_copy(x_hbm.at[pl.ds(tbase + i * L, L)], rows)
            dv = dflat[pl.ds(i * L, L)]
            pltpu.async_copy(rows, xs_hbm.at[dv], sem).wait()
            return 0

        lax.fori_loop(0, nchunks, body, 0)

    return k(flat_x, experts, ranks, poff)


def _combine(outs, dest, wts, t, d):
    tpw = t // NW
    nchunks = tpw // L

    mesh = plsc.VectorSubcoreMesh(core_axis_name="c", subcore_axis_name="s")

    @functools.partial(
        pl.kernel,
        mesh=mesh,
        compiler_params=pltpu.CompilerParams(needs_layout_passes=False),
        out_type=jax.ShapeDtypeStruct((t, d), jnp.float32),
        scratch_types=[
            pltpu.VMEM((tpw,), jnp.int32),
            pltpu.VMEM((tpw,), jnp.int32),
            pltpu.VMEM((tpw,), jnp.float32),
            pltpu.VMEM((tpw,), jnp.float32),
            pltpu.VMEM((L, 1024), jnp.float32),
            pltpu.VMEM((L, 1024), jnp.float32),
            pltpu.VMEM((L, 1024), jnp.float32),
            pltpu.SemaphoreType.DMA,
            pltpu.SemaphoreType.DMA,
        ],
    )
    def k(outs_hbm, dest_hbm, w_hbm, y_hbm,
          d0, d1, w0, w1, r0, r1, yb, sem0, sem1):
        wid = lax.axis_index("s") * NC + lax.axis_index("c")
        tb = wid * tpw
        chunk = tb // TB
        pos0 = PB * chunk + (tb % TB)
        pos1 = pos0 + TB

        pltpu.sync_copy(dest_hbm.at[pl.ds(pos0, tpw)], d0)
        pltpu.sync_copy(dest_hbm.at[pl.ds(pos1, tpw)], d1)
        pltpu.sync_copy(w_hbm.at[pl.ds(pos0, tpw)], w0)
        pltpu.sync_copy(w_hbm.at[pl.ds(pos1, tpw)], w1)

        def body(i, _):
            di0 = d0[pl.ds(i * L, L)]
            di1 = d1[pl.ds(i * L, L)]
            cp0 = pltpu.async_copy(outs_hbm.at[di0], r0, sem0)
            cp1 = pltpu.async_copy(outs_hbm.at[di1], r1, sem1)
            cp0.wait()
            cp1.wait()

            def tokbody(j, _):
                idx = jnp.full((L,), i * L + j, jnp.int32)
                wa = plsc.load_gather(w0, [idx])
                wb = plsc.load_gather(w1, [idx])
                for cc in range(d // L):
                    yb[j, pl.ds(cc * L, L)] = (wa * r0[j, pl.ds(cc * L, L)]
                                               + wb * r1[j, pl.ds(cc * L, L)])
                return 0

            lax.fori_loop(0, L, tokbody, 0)
            pltpu.sync_copy(yb, y_hbm.at[pl.ds(tb + i * L, L)])
            return 0

        lax.fori_loop(0, nchunks, body, 0)

    return k(outs, dest, wts)


def _ffn_body(be_ref, xs_ref, gw_ref, uw_ref, dw_ref, out_ref):
    j = pl.program_id(1)
    x = xs_ref[...]
    g = jax.lax.dot_general(x, gw_ref[0], (((1,), (1,)), ((), ())),
                            preferred_element_type=jnp.float32)
    u = jax.lax.dot_general(x, uw_ref[0], (((1,), (1,)), ((), ())),
                            preferred_element_type=jnp.float32)
    h = g * jax.nn.sigmoid(g) * u
    o = jax.lax.dot_general(h, dw_ref[0], (((1,), (1,)), ((), ())),
                            preferred_element_type=jnp.float32)

    @pl.when(j == 0)
    def _():
        out_ref[...] = o

    @pl.when(j > 0)
    def _():
        out_ref[...] += o


def _grouped_ffn(xs, gate_w, up_w, down_w, block_expert):
    pt, d = xs.shape
    e, di, _ = gate_w.shape
    nb = pt // BLK
    it = di // DI_T
    grid_spec = pltpu.PrefetchScalarGridSpec(
        num_scalar_prefetch=1,
        grid=(nb, it),
        in_specs=[
            pl.BlockSpec((BLK, d), lambda i, j, be: (i, 0)),
            pl.BlockSpec((1, DI_T, d), lambda i, j, be: (be[i], j, 0)),
            pl.BlockSpec((1, DI_T, d), lambda i, j, be: (be[i], j, 0)),
            pl.BlockSpec((1, d, DI_T), lambda i, j, be: (be[i], 0, j)),
        ],
        out_specs=pl.BlockSpec((BLK, d), lambda i, j, be: (i, 0)),
    )
    return pl.pallas_call(
        _ffn_body,
        grid_spec=grid_spec,
        out_shape=jax.ShapeDtypeStruct((pt, d), jnp.float32),
    )(block_expert, xs, gate_w, up_w, down_w)


def kernel(x, router_w, router_b, gate_w, up_w, down_w):
    b, s, d = x.shape
    t = b * s
    tk = t * K
    pt = tk + E * BLK
    nb = pt // BLK

    flat_x = x.reshape(t, d)

    experts, ranks, wts, counts = _router(flat_x, router_w, router_b)

    padded = ((counts + BLK - 1) // BLK) * BLK
    pend = jnp.cumsum(padded)
    poff = (pend - padded).astype(jnp.int32)
    block_expert = jnp.minimum(
        jnp.searchsorted(pend, jnp.arange(nb, dtype=jnp.int32) * BLK,
                         side='right'),
        E - 1).astype(jnp.int32)

    xs, dest = _dispatch(flat_x, experts, ranks, poff, pt)

    outs = _grouped_ffn(xs, gate_w, up_w, down_w, block_expert)

    y = _combine(outs, dest, wts, t, d)
    return y.reshape(b, s, d)


# speedup vs baseline: 1.4503x; 1.1506x over previous
import functools

import jax
import jax.numpy as jnp
from jax import lax
from jax.experimental import pallas as pl
from jax.experimental.pallas import tpu as pltpu
from jax.experimental.pallas import tpu_sc as plsc

E = 8
K = 2
BLK = 256
DI_T = 1024
TB = 512
PB = TB * K

NC = 2
NS = 16
NW = NC * NS
L = 16


def _router_body(x_ref, rw_ref, rb_ref, e_ref, r_ref, w_ref, cnt_ref, carry):
    c = pl.program_id(0)

    @pl.when(c == 0)
    def _():
        carry[...] = jnp.zeros_like(carry)

    lg = jax.lax.dot_general(rw_ref[...].astype(jnp.bfloat16),
                             x_ref[...].astype(jnp.bfloat16),
                             (((1,), (1,)), ((), ())),
                             preferred_element_type=jnp.float32)
    lg = lg + rb_ref[...].reshape(E, 1)
    m = jnp.max(lg, axis=0, keepdims=True)
    q = jnp.exp(lg - m)
    p = q / jnp.sum(q, axis=0, keepdims=True)
    iota = jax.lax.broadcasted_iota(jnp.int32, (E, TB), 0)
    m1 = jnp.max(p, axis=0, keepdims=True)
    i1 = jnp.min(jnp.where(p == m1, iota, E), axis=0, keepdims=True)
    masked = jnp.where(iota == i1, -1.0, p)
    m2 = jnp.max(masked, axis=0, keepdims=True)
    i2 = jnp.min(jnp.where((masked == m2) & (iota != i1), iota, E),
                 axis=0, keepdims=True)
    s = m1 + m2
    w1 = m1 / s
    w2 = m2 / s

    oh = jnp.concatenate([(iota == i1), (iota == i2)],
                         axis=1).astype(jnp.float32)
    pr = jax.lax.broadcasted_iota(jnp.int32, (PB, PB), 0)
    pc = jax.lax.broadcasted_iota(jnp.int32, (PB, PB), 1)
    ltu = (pr < pc).astype(jnp.float32)
    cum = jax.lax.dot_general(oh, ltu, (((1,), (0,)), ((), ())),
                              preferred_element_type=jnp.float32,
                              precision=jax.lax.Precision.HIGHEST)
    rank = (jnp.sum(oh * cum, axis=0, keepdims=True)
            + jax.lax.dot_general(carry[...], oh, (((1,), (0,)), ((), ())),
                                  preferred_element_type=jnp.float32,
                                  precision=jax.lax.Precision.HIGHEST))

    evec = jnp.concatenate([i1, i2], axis=1)
    wvec = jnp.concatenate([w1, w2], axis=1)
    e_ref[...] = evec.reshape(PB)
    r_ref[...] = rank.astype(jnp.int32).reshape(PB)
    w_ref[...] = wvec.reshape(PB)
    carry[...] += jax.lax.dot_general(
        jnp.ones((1, PB), jnp.float32), oh, (((1,), (1,)), ((), ())),
        preferred_element_type=jnp.float32,
        precision=jax.lax.Precision.HIGHEST)
    cnt_ref[...] = carry[...].astype(jnp.int32).reshape(E)


def _router(flat_x, router_w, router_b):
    t, d = flat_x.shape
    nc = t // TB
    tk = t * K
    grid_spec = pltpu.PrefetchScalarGridSpec(
        num_scalar_prefetch=0,
        grid=(nc,),
        in_specs=[
            pl.BlockSpec((TB, d), lambda c: (c, 0)),
            pl.BlockSpec((E, d), lambda c: (0, 0)),
            pl.BlockSpec((1, E), lambda c: (0, 0)),
        ],
        out_specs=[
            pl.BlockSpec((PB,), lambda c: (c,)),
            pl.BlockSpec((PB,), lambda c: (c,)),
            pl.BlockSpec((PB,), lambda c: (c,)),
            pl.BlockSpec((E,), lambda c: (0,)),
        ],
        scratch_shapes=[pltpu.VMEM((1, E), jnp.float32)],
    )
    return pl.pallas_call(
        _router_body,
        grid_spec=grid_spec,
        out_shape=[
            jax.ShapeDtypeStruct((tk,), jnp.int32),
            jax.ShapeDtypeStruct((tk,), jnp.int32),
            jax.ShapeDtypeStruct((tk,), jnp.float32),
            jax.ShapeDtypeStruct((E,), jnp.int32),
        ],
    )(flat_x, router_w, router_b.reshape(1, E))


def _dispatch(flat_x, experts, ranks, poff, pt):
    t, d = flat_x.shape
    tk = t * K
    ppw = tk // NW
    nchunks = ppw // L

    mesh = plsc.VectorSubcoreMesh(core_axis_name="c", subcore_axis_name="s")

    @functools.partial(
        pl.kernel,
        mesh=mesh,
        compiler_params=pltpu.CompilerParams(needs_layout_passes=False),
        out_type=[
            jax.ShapeDtypeStruct((pt, d), jnp.float32),
            jax.ShapeDtypeStruct((tk,), jnp.int32),
        ],
        scratch_types=[
            pltpu.VMEM((ppw,), jnp.int32),
            pltpu.VMEM((ppw,), jnp.int32),
            pltpu.VMEM((L,), jnp.int32),
            pltpu.VMEM((ppw,), jnp.int32),
            pltpu.VMEM((L, 1024), jnp.float32),
            pltpu.SemaphoreType.DMA,
        ],
    )
    def k(x_hbm, e_hbm, r_hbm, poff_hbm, xs_hbm, dest_hbm,
          evm, rvm, pvm, dflat, rows, sem):
        wid = lax.axis_index("s") * NC + lax.axis_index("c")
        p0 = wid * ppw
        chunk = p0 // PB
        tbase = TB * chunk + (p0 % TB)

        pltpu.sync_copy(e_hbm.at[pl.ds(p0, ppw)], evm)
        pltpu.sync_copy(r_hbm.at[pl.ds(p0, ppw)], rvm)
        pltpu.sync_copy(poff_hbm, pvm.at[pl.ds(0, E)])

        for i in range(nchunks):
            ev = evm[pl.ds(i * L, L)]
            rv = rvm[pl.ds(i * L, L)]
            dflat[pl.ds(i * L, L)] = plsc.load_gather(pvm, [ev]) + rv

        pltpu.sync_copy(dflat, dest_hbm.at[pl.ds(p0, ppw)])

        def body(i, _):
            pltpu.sync_copy(x_hbm.at[pl.ds(tbase + i * L, L)], rows)
            dv = dflat[pl.ds(i * L, L)]
            pltpu.async_copy(rows, xs_hbm.at[dv], sem).wait()
            return 0

        lax.fori_loop(0, nchunks, body, 0)

    return k(flat_x, experts, ranks, poff)


def _combine(outs, dest, wts, t, d):
    tpw = t // NW
    nchunks = tpw // L

    mesh = plsc.VectorSubcoreMesh(core_axis_name="c", subcore_axis_name="s")

    @functools.partial(
        pl.kernel,
        mesh=mesh,
        compiler_params=pltpu.CompilerParams(needs_layout_passes=False),
        out_type=jax.ShapeDtypeStruct((t, d), jnp.float32),
        scratch_types=[
            pltpu.VMEM((tpw,), jnp.int32),
            pltpu.VMEM((tpw,), jnp.int32),
            pltpu.VMEM((tpw,), jnp.float32),
            pltpu.VMEM((tpw,), jnp.float32),
            pltpu.VMEM((L, 1024), jnp.float32),
            pltpu.VMEM((L, 1024), jnp.float32),
            pltpu.VMEM((L, 1024), jnp.float32),
            pltpu.SemaphoreType.DMA,
            pltpu.SemaphoreType.DMA,
        ],
    )
    def k(outs_hbm, dest_hbm, w_hbm, y_hbm,
          d0, d1, w0, w1, r0, r1, yb, sem0, sem1):
        wid = lax.axis_index("s") * NC + lax.axis_index("c")
        tb = wid * tpw
        chunk = tb // TB
        pos0 = PB * chunk + (tb % TB)
        pos1 = pos0 + TB

        pltpu.sync_copy(dest_hbm.at[pl.ds(pos0, tpw)], d0)
        pltpu.sync_copy(dest_hbm.at[pl.ds(pos1, tpw)], d1)
        pltpu.sync_copy(w_hbm.at[pl.ds(pos0, tpw)], w0)
        pltpu.sync_copy(w_hbm.at[pl.ds(pos1, tpw)], w1)

        def body(i, _):
            di0 = d0[pl.ds(i * L, L)]
            di1 = d1[pl.ds(i * L, L)]
            cp0 = pltpu.async_copy(outs_hbm.at[di0], r0, sem0)
            cp1 = pltpu.async_copy(outs_hbm.at[di1], r1, sem1)
            cp0.wait()
            cp1.wait()

            def tokbody(j, _):
                idx = jnp.full((L,), i * L + j, jnp.int32)
                wa = plsc.load_gather(w0, [idx])
                wb = plsc.load_gather(w1, [idx])
                for cc in range(d // L):
                    yb[j, pl.ds(cc * L, L)] = (wa * r0[j, pl.ds(cc * L, L)]
                                               + wb * r1[j, pl.ds(cc * L, L)])
                return 0

            lax.fori_loop(0, L, tokbody, 0)
            pltpu.sync_copy(yb, y_hbm.at[pl.ds(tb + i * L, L)])
            return 0

        lax.fori_loop(0, nchunks, body, 0)

    return k(outs, dest, wts)


def _ffn_body(be_ref, xs_ref, gw_ref, uw_ref, dw_ref, out_ref):
    x = xs_ref[...].astype(jnp.bfloat16)
    g = jax.lax.dot_general(x, gw_ref[0], (((1,), (1,)), ((), ())),
                            preferred_element_type=jnp.float32)
    u = jax.lax.dot_general(x, uw_ref[0], (((1,), (1,)), ((), ())),
                            preferred_element_type=jnp.float32)
    h = (g * jax.nn.sigmoid(g) * u).astype(jnp.bfloat16)
    out_ref[...] = jax.lax.dot_general(h, dw_ref[0], (((1,), (1,)), ((), ())),
                                       preferred_element_type=jnp.float32)


def _grouped_ffn(xs, gate_w, up_w, down_w, block_expert):
    pt, d = xs.shape
    e, di, _ = gate_w.shape
    nb = pt // BLK
    grid_spec = pltpu.PrefetchScalarGridSpec(
        num_scalar_prefetch=1,
        grid=(nb,),
        in_specs=[
            pl.BlockSpec((BLK, d), lambda i, be: (i, 0)),
            pl.BlockSpec((1, di, d), lambda i, be: (be[i], 0, 0)),
            pl.BlockSpec((1, di, d), lambda i, be: (be[i], 0, 0)),
            pl.BlockSpec((1, d, di), lambda i, be: (be[i], 0, 0)),
        ],
        out_specs=pl.BlockSpec((BLK, d), lambda i, be: (i, 0)),
    )
    return pl.pallas_call(
        _ffn_body,
        grid_spec=grid_spec,
        out_shape=jax.ShapeDtypeStruct((pt, d), jnp.float32),
    )(block_expert, xs, gate_w.astype(jnp.bfloat16),
      up_w.astype(jnp.bfloat16), down_w.astype(jnp.bfloat16))


def kernel(x, router_w, router_b, gate_w, up_w, down_w):
    b, s, d = x.shape
    t = b * s
    tk = t * K
    pt = tk + E * BLK
    nb = pt // BLK

    flat_x = x.reshape(t, d)

    experts, ranks, wts, counts = _router(flat_x, router_w, router_b)

    padded = ((counts + BLK - 1) // BLK) * BLK
    pend = jnp.cumsum(padded)
    poff = (pend - padded).astype(jnp.int32)
    block_expert = jnp.minimum(
        jnp.searchsorted(pend, jnp.arange(nb, dtype=jnp.int32) * BLK,
                         side='right'),
        E - 1).astype(jnp.int32)

    xs, dest = _dispatch(flat_x, experts, ranks, poff, pt)

    outs = _grouped_ffn(xs, gate_w, up_w, down_w, block_expert)

    y = _combine(outs, dest, wts, t, d)
    return y.reshape(b, s, d)


# speedup vs baseline: 1.7775x; 1.2256x over previous
import functools

import jax
import jax.numpy as jnp
from jax import lax
from jax.experimental import pallas as pl
from jax.experimental.pallas import tpu as pltpu
from jax.experimental.pallas import tpu_sc as plsc

E = 8
K = 2
BLK = 256
DI_T = 1024
TB = 512
PB = TB * K

NC = 2
NS = 16
NW = NC * NS
L = 16


def _router_body(x_ref, rw_ref, rb_ref, e_ref, r_ref, w_ref, cnt_ref, carry):
    c = pl.program_id(0)

    @pl.when(c == 0)
    def _():
        carry[...] = jnp.zeros_like(carry)

    lg = jax.lax.dot_general(rw_ref[...].astype(jnp.bfloat16),
                             x_ref[...].astype(jnp.bfloat16),
                             (((1,), (1,)), ((), ())),
                             preferred_element_type=jnp.float32)
    lg = lg + rb_ref[...].reshape(E, 1)
    m = jnp.max(lg, axis=0, keepdims=True)
    q = jnp.exp(lg - m)
    p = q / jnp.sum(q, axis=0, keepdims=True)
    iota = jax.lax.broadcasted_iota(jnp.int32, (E, TB), 0)
    m1 = jnp.max(p, axis=0, keepdims=True)
    i1 = jnp.min(jnp.where(p == m1, iota, E), axis=0, keepdims=True)
    masked = jnp.where(iota == i1, -1.0, p)
    m2 = jnp.max(masked, axis=0, keepdims=True)
    i2 = jnp.min(jnp.where((masked == m2) & (iota != i1), iota, E),
                 axis=0, keepdims=True)
    s = m1 + m2
    w1 = m1 / s
    w2 = m2 / s

    oh = jnp.concatenate([(iota == i1), (iota == i2)],
                         axis=1).astype(jnp.float32)
    pr = jax.lax.broadcasted_iota(jnp.int32, (PB, PB), 0)
    pc = jax.lax.broadcasted_iota(jnp.int32, (PB, PB), 1)
    ltu = (pr < pc).astype(jnp.float32)
    cum = jax.lax.dot_general(oh, ltu, (((1,), (0,)), ((), ())),
                              preferred_element_type=jnp.float32,
                              precision=jax.lax.Precision.HIGHEST)
    rank = (jnp.sum(oh * cum, axis=0, keepdims=True)
            + jax.lax.dot_general(carry[...], oh, (((1,), (0,)), ((), ())),
                                  preferred_element_type=jnp.float32,
                                  precision=jax.lax.Precision.HIGHEST))

    evec = jnp.concatenate([i1, i2], axis=1)
    wvec = jnp.concatenate([w1, w2], axis=1)
    e_ref[...] = evec.reshape(PB)
    r_ref[...] = rank.astype(jnp.int32).reshape(PB)
    w_ref[...] = wvec.reshape(PB)
    carry[...] += jax.lax.dot_general(
        jnp.ones((1, PB), jnp.float32), oh, (((1,), (1,)), ((), ())),
        preferred_element_type=jnp.float32,
        precision=jax.lax.Precision.HIGHEST)
    cnt_ref[...] = carry[...].astype(jnp.int32).reshape(E)


def _router(flat_x, router_w, router_b):
    t, d = flat_x.shape
    nc = t // TB
    tk = t * K
    grid_spec = pltpu.PrefetchScalarGridSpec(
        num_scalar_prefetch=0,
        grid=(nc,),
        in_specs=[
            pl.BlockSpec((TB, d), lambda c: (c, 0)),
            pl.BlockSpec((E, d), lambda c: (0, 0)),
            pl.BlockSpec((1, E), lambda c: (0, 0)),
        ],
        out_specs=[
            pl.BlockSpec((PB,), lambda c: (c,)),
            pl.BlockSpec((PB,), lambda c: (c,)),
            pl.BlockSpec((PB,), lambda c: (c,)),
            pl.BlockSpec((E,), lambda c: (0,)),
        ],
        scratch_shapes=[pltpu.VMEM((1, E), jnp.float32)],
    )
    return pl.pallas_call(
        _router_body,
        grid_spec=grid_spec,
        out_shape=[
            jax.ShapeDtypeStruct((tk,), jnp.int32),
            jax.ShapeDtypeStruct((tk,), jnp.int32),
            jax.ShapeDtypeStruct((tk,), jnp.float32),
            jax.ShapeDtypeStruct((E,), jnp.int32),
        ],
    )(flat_x, router_w, router_b.reshape(1, E))


def _dispatch(flat_x, experts, ranks, poff, pt):
    t, d = flat_x.shape
    tk = t * K
    ppw = tk // NW
    nchunks = ppw // L

    mesh = plsc.VectorSubcoreMesh(core_axis_name="c", subcore_axis_name="s")

    @functools.partial(
        pl.kernel,
        mesh=mesh,
        compiler_params=pltpu.CompilerParams(needs_layout_passes=False),
        out_type=[
            jax.ShapeDtypeStruct((pt, d), jnp.float32),
            jax.ShapeDtypeStruct((tk,), jnp.int32),
        ],
        scratch_types=[
            pltpu.VMEM((ppw,), jnp.int32),
            pltpu.VMEM((ppw,), jnp.int32),
            pltpu.VMEM((L,), jnp.int32),
            pltpu.VMEM((ppw,), jnp.int32),
            pltpu.VMEM((L, 1024), jnp.float32),
            pltpu.SemaphoreType.DMA,
        ],
    )
    def k(x_hbm, e_hbm, r_hbm, poff_hbm, xs_hbm, dest_hbm,
          evm, rvm, pvm, dflat, rows, sem):
        wid = lax.axis_index("s") * NC + lax.axis_index("c")
        p0 = wid * ppw
        chunk = p0 // PB
        tbase = TB * chunk + (p0 % TB)

        pltpu.sync_copy(e_hbm.at[pl.ds(p0, ppw)], evm)
        pltpu.sync_copy(r_hbm.at[pl.ds(p0, ppw)], rvm)
        pltpu.sync_copy(poff_hbm, pvm.at[pl.ds(0, E)])

        for i in range(nchunks):
            ev = evm[pl.ds(i * L, L)]
            rv = rvm[pl.ds(i * L, L)]
            dflat[pl.ds(i * L, L)] = plsc.load_gather(pvm, [ev]) + rv

        pltpu.sync_copy(dflat, dest_hbm.at[pl.ds(p0, ppw)])

        def body(i, _):
            pltpu.sync_copy(x_hbm.at[pl.ds(tbase + i * L, L)], rows)
            dv = dflat[pl.ds(i * L, L)]
            pltpu.async_copy(rows, xs_hbm.at[dv], sem).wait()
            return 0

        lax.fori_loop(0, nchunks, body, 0)

    return k(flat_x, experts, ranks, poff)


def _combine(outs, dest, wts, t, d):
    tpw = t // NW
    nchunks = tpw // L

    mesh = plsc.VectorSubcoreMesh(core_axis_name="c", subcore_axis_name="s")

    @functools.partial(
        pl.kernel,
        mesh=mesh,
        compiler_params=pltpu.CompilerParams(needs_layout_passes=False),
        out_type=jax.ShapeDtypeStruct((t, d), jnp.float32),
        scratch_types=[
            pltpu.VMEM((tpw,), jnp.int32),
            pltpu.VMEM((tpw,), jnp.int32),
            pltpu.VMEM((tpw,), jnp.float32),
            pltpu.VMEM((tpw,), jnp.float32),
            pltpu.VMEM((L, 1024), jnp.float32),
            pltpu.VMEM((L, 1024), jnp.float32),
            pltpu.VMEM((L, 1024), jnp.float32),
            pltpu.SemaphoreType.DMA,
            pltpu.SemaphoreType.DMA,
        ],
    )
    def k(outs_hbm, dest_hbm, w_hbm, y_hbm,
          d0, d1, w0, w1, r0, r1, yb, sem0, sem1):
        wid = lax.axis_index("s") * NC + lax.axis_index("c")
        tb = wid * tpw
        chunk = tb // TB
        pos0 = PB * chunk + (tb % TB)
        pos1 = pos0 + TB

        pltpu.sync_copy(dest_hbm.at[pl.ds(pos0, tpw)], d0)
        pltpu.sync_copy(dest_hbm.at[pl.ds(pos1, tpw)], d1)
        pltpu.sync_copy(w_hbm.at[pl.ds(pos0, tpw)], w0)
        pltpu.sync_copy(w_hbm.at[pl.ds(pos1, tpw)], w1)

        def body(i, _):
            di0 = d0[pl.ds(i * L, L)]
            di1 = d1[pl.ds(i * L, L)]
            cp0 = pltpu.async_copy(outs_hbm.at[di0], r0, sem0)
            cp1 = pltpu.async_copy(outs_hbm.at[di1], r1, sem1)
            cp0.wait()
            cp1.wait()

            def tokbody(j, _):
                idx = jnp.full((L,), i * L + j, jnp.int32)
                wa = plsc.load_gather(w0, [idx])
                wb = plsc.load_gather(w1, [idx])
                for cc in range(d // L):
                    yb[j, pl.ds(cc * L, L)] = (wa * r0[j, pl.ds(cc * L, L)]
                                               + wb * r1[j, pl.ds(cc * L, L)])
                return 0

            lax.fori_loop(0, L, tokbody, 0)
            pltpu.sync_copy(yb, y_hbm.at[pl.ds(tb + i * L, L)])
            return 0

        lax.fori_loop(0, nchunks, body, 0)

    return k(outs, dest, wts)


def _ffn_body(be_ref, xs_ref, gw_ref, uw_ref, dw_ref, out_ref):
    x = xs_ref[...]
    g = jax.lax.dot_general(x, gw_ref[0], (((1,), (1,)), ((), ())),
                            preferred_element_type=jnp.float32)
    u = jax.lax.dot_general(x, uw_ref[0], (((1,), (1,)), ((), ())),
                            preferred_element_type=jnp.float32)
    h = (g * jax.nn.sigmoid(g) * u).astype(jnp.bfloat16)
    out_ref[...] = jax.lax.dot_general(h, dw_ref[0], (((1,), (1,)), ((), ())),
                                       preferred_element_type=jnp.float32)


def _grouped_ffn(xs, gate_w, up_w, down_w, block_expert):
    pt, d = xs.shape
    e, di, _ = gate_w.shape
    nb = pt // BLK
    grid_spec = pltpu.PrefetchScalarGridSpec(
        num_scalar_prefetch=1,
        grid=(nb,),
        in_specs=[
            pl.BlockSpec((BLK, d), lambda i, be: (i, 0)),
            pl.BlockSpec((1, di, d), lambda i, be: (be[i], 0, 0)),
            pl.BlockSpec((1, di, d), lambda i, be: (be[i], 0, 0)),
            pl.BlockSpec((1, d, di), lambda i, be: (be[i], 0, 0)),
        ],
        out_specs=pl.BlockSpec((BLK, d), lambda i, be: (i, 0)),
    )
    return pl.pallas_call(
        _ffn_body,
        grid_spec=grid_spec,
        out_shape=jax.ShapeDtypeStruct((pt, d), jnp.float32),
    )(block_expert, xs, gate_w, up_w, down_w)


def kernel(x, router_w, router_b, gate_w, up_w, down_w):
    b, s, d = x.shape
    t = b * s
    tk = t * K
    pt = tk + E * BLK
    nb = pt // BLK

    flat_x = x.reshape(t, d)

    experts, ranks, wts, counts = _router(flat_x, router_w, router_b)

    padded = ((counts + BLK - 1) // BLK) * BLK
    pend = jnp.cumsum(padded)
    poff = (pend - padded).astype(jnp.int32)
    block_expert = jnp.minimum(
        jnp.searchsorted(pend, jnp.arange(nb, dtype=jnp.int32) * BLK,
                         side='right'),
        E - 1).astype(jnp.int32)

    xs, dest = _dispatch(flat_x, experts, ranks, poff, pt)

    outs = _grouped_ffn(xs, gate_w, up_w, down_w, block_expert)

    y = _combine(outs, dest, wts, t, d)
    return y.reshape(b, s, d)


# speedup vs baseline: 1.9026x; 1.0704x over previous
import functools

import jax
import jax.numpy as jnp
from jax import lax
from jax.experimental import pallas as pl
from jax.experimental.pallas import tpu as pltpu
from jax.experimental.pallas import tpu_sc as plsc

E = 8
K = 2
BLK = 256
DI_T = 1024
TB = 512
PB = TB * K

NC = 2
NS = 16
NW = NC * NS
L = 16


def _router_body(x_ref, rw_ref, rb_ref, e_ref, r_ref, w_ref, cnt_ref, carry):
    c = pl.program_id(0)

    @pl.when(c == 0)
    def _():
        carry[...] = jnp.zeros_like(carry)

    lg = jax.lax.dot_general(rw_ref[...].astype(jnp.bfloat16),
                             x_ref[...].astype(jnp.bfloat16),
                             (((1,), (1,)), ((), ())),
                             preferred_element_type=jnp.float32)
    lg = lg + rb_ref[...].reshape(E, 1)
    m = jnp.max(lg, axis=0, keepdims=True)
    q = jnp.exp(lg - m)
    p = q / jnp.sum(q, axis=0, keepdims=True)
    iota = jax.lax.broadcasted_iota(jnp.int32, (E, TB), 0)
    m1 = jnp.max(p, axis=0, keepdims=True)
    i1 = jnp.min(jnp.where(p == m1, iota, E), axis=0, keepdims=True)
    masked = jnp.where(iota == i1, -1.0, p)
    m2 = jnp.max(masked, axis=0, keepdims=True)
    i2 = jnp.min(jnp.where((masked == m2) & (iota != i1), iota, E),
                 axis=0, keepdims=True)
    s = m1 + m2
    w1 = m1 / s
    w2 = m2 / s

    oh = jnp.concatenate([(iota == i1), (iota == i2)],
                         axis=1).astype(jnp.float32)
    pr = jax.lax.broadcasted_iota(jnp.int32, (PB, PB), 0)
    pc = jax.lax.broadcasted_iota(jnp.int32, (PB, PB), 1)
    ltu = (pr < pc).astype(jnp.float32)
    cum = jax.lax.dot_general(oh, ltu, (((1,), (0,)), ((), ())),
                              preferred_element_type=jnp.float32,
                              precision=jax.lax.Precision.HIGHEST)
    rank = (jnp.sum(oh * cum, axis=0, keepdims=True)
            + jax.lax.dot_general(carry[...], oh, (((1,), (0,)), ((), ())),
                                  preferred_element_type=jnp.float32,
                                  precision=jax.lax.Precision.HIGHEST))

    evec = jnp.concatenate([i1, i2], axis=1)
    wvec = jnp.concatenate([w1, w2], axis=1)
    e_ref[...] = evec.reshape(PB)
    r_ref[...] = rank.astype(jnp.int32).reshape(PB)
    w_ref[...] = wvec.reshape(PB)
    carry[...] += jax.lax.dot_general(
        jnp.ones((1, PB), jnp.float32), oh, (((1,), (1,)), ((), ())),
        preferred_element_type=jnp.float32,
        precision=jax.lax.Precision.HIGHEST)
    cnt_ref[...] = carry[...].astype(jnp.int32).reshape(E)


def _router(flat_x, router_w, router_b):
    t, d = flat_x.shape
    nc = t // TB
    tk = t * K
    grid_spec = pltpu.PrefetchScalarGridSpec(
        num_scalar_prefetch=0,
        grid=(nc,),
        in_specs=[
            pl.BlockSpec((TB, d), lambda c: (c, 0)),
            pl.BlockSpec((E, d), lambda c: (0, 0)),
            pl.BlockSpec((1, E), lambda c: (0, 0)),
        ],
        out_specs=[
            pl.BlockSpec((PB,), lambda c: (c,)),
            pl.BlockSpec((PB,), lambda c: (c,)),
            pl.BlockSpec((PB,), lambda c: (c,)),
            pl.BlockSpec((E,), lambda c: (0,)),
        ],
        scratch_shapes=[pltpu.VMEM((1, E), jnp.float32)],
    )
    return pl.pallas_call(
        _router_body,
        grid_spec=grid_spec,
        out_shape=[
            jax.ShapeDtypeStruct((tk,), jnp.int32),
            jax.ShapeDtypeStruct((tk,), jnp.int32),
            jax.ShapeDtypeStruct((tk,), jnp.float32),
            jax.ShapeDtypeStruct((E,), jnp.int32),
        ],
    )(flat_x, router_w, router_b.reshape(1, E))


def _dispatch(flat_x, experts, ranks, poff, pt):
    t, d = flat_x.shape
    tk = t * K
    ppw = tk // NW
    nchunks = ppw // L

    mesh = plsc.VectorSubcoreMesh(core_axis_name="c", subcore_axis_name="s")

    nbuf = 6
    half = nbuf // 2

    @functools.partial(
        pl.kernel,
        mesh=mesh,
        compiler_params=pltpu.CompilerParams(needs_layout_passes=False),
        out_type=[
            jax.ShapeDtypeStruct((pt, d), jnp.float32),
            jax.ShapeDtypeStruct((tk,), jnp.int32),
        ],
        scratch_types=[
            pltpu.VMEM((ppw,), jnp.int32),
            pltpu.VMEM((ppw,), jnp.int32),
            pltpu.VMEM((L,), jnp.int32),
            pltpu.VMEM((ppw,), jnp.int32),
            [pltpu.VMEM((L, 1024), jnp.float32) for _ in range(nbuf)],
            [pltpu.SemaphoreType.DMA for _ in range(nbuf)],
            [pltpu.SemaphoreType.DMA for _ in range(nbuf)],
        ],
    )
    def k(x_hbm, e_hbm, r_hbm, poff_hbm, xs_hbm, dest_hbm,
          evm, rvm, pvm, dflat, rows, rsem, wsem):
        wid = lax.axis_index("s") * NC + lax.axis_index("c")
        p0 = wid * ppw
        chunk = p0 // PB
        tbase = TB * chunk + (p0 % TB)

        pltpu.sync_copy(e_hbm.at[pl.ds(p0, ppw)], evm)
        pltpu.sync_copy(r_hbm.at[pl.ds(p0, ppw)], rvm)
        pltpu.sync_copy(poff_hbm, pvm.at[pl.ds(0, E)])

        for i in range(nchunks):
            ev = evm[pl.ds(i * L, L)]
            rv = rvm[pl.ds(i * L, L)]
            dflat[pl.ds(i * L, L)] = plsc.load_gather(pvm, [ev]) + rv

        pltpu.sync_copy(dflat, dest_hbm.at[pl.ds(p0, ppw)])

        rd = [None] * nbuf
        sc = [None] * nbuf
        for j in range(half):
            rd[j] = pltpu.async_copy(x_hbm.at[pl.ds(tbase + j * L, L)],
                                     rows[j], rsem[j])
        for i in range(nchunks):
            b = i % nbuf
            j = i + half
            if half <= j < nchunks:
                bj = j % nbuf
                if sc[bj] is not None:
                    sc[bj].wait()
                    sc[bj] = None
                rd[bj] = pltpu.async_copy(x_hbm.at[pl.ds(tbase + j * L, L)],
                                          rows[bj], rsem[bj])
            rd[b].wait()
            dv = dflat[pl.ds(i * L, L)]
            sc[b] = pltpu.async_copy(rows[b], xs_hbm.at[dv], wsem[b])
        for b in range(nbuf):
            if sc[b] is not None:
                sc[b].wait()

    return k(flat_x, experts, ranks, poff)


def _combine(outs, dest, wts, t, d):
    tpw = t // NW
    nchunks = tpw // L

    mesh = plsc.VectorSubcoreMesh(core_axis_name="c", subcore_axis_name="s")

    @functools.partial(
        pl.kernel,
        mesh=mesh,
        compiler_params=pltpu.CompilerParams(needs_layout_passes=False),
        out_type=jax.ShapeDtypeStruct((t, d), jnp.float32),
        scratch_types=[
            pltpu.VMEM((tpw,), jnp.int32),
            pltpu.VMEM((tpw,), jnp.int32),
            pltpu.VMEM((tpw,), jnp.float32),
            pltpu.VMEM((tpw,), jnp.float32),
            [pltpu.VMEM((L, 1024), jnp.float32) for _ in range(2)],
            [pltpu.VMEM((L, 1024), jnp.float32) for _ in range(2)],
            [pltpu.VMEM((L, 1024), jnp.float32) for _ in range(2)],
            [pltpu.SemaphoreType.DMA for _ in range(2)],
            [pltpu.SemaphoreType.DMA for _ in range(2)],
            [pltpu.SemaphoreType.DMA for _ in range(2)],
        ],
    )
    def k(outs_hbm, dest_hbm, w_hbm, y_hbm,
          d0, d1, w0, w1, r0, r1, yb, sem0, sem1, ysem):
        wid = lax.axis_index("s") * NC + lax.axis_index("c")
        tb = wid * tpw
        chunk = tb // TB
        pos0 = PB * chunk + (tb % TB)
        pos1 = pos0 + TB

        pltpu.sync_copy(dest_hbm.at[pl.ds(pos0, tpw)], d0)
        pltpu.sync_copy(dest_hbm.at[pl.ds(pos1, tpw)], d1)
        pltpu.sync_copy(w_hbm.at[pl.ds(pos0, tpw)], w0)
        pltpu.sync_copy(w_hbm.at[pl.ds(pos1, tpw)], w1)

        def gathers(i, b):
            cp0 = pltpu.async_copy(outs_hbm.at[d0[pl.ds(i * L, L)]],
                                   r0[b], sem0[b])
            cp1 = pltpu.async_copy(outs_hbm.at[d1[pl.ds(i * L, L)]],
                                   r1[b], sem1[b])
            return cp0, cp1

        g = [gathers(0, 0), gathers(1, 1)]
        wb = [None, None]
        for i in range(nchunks):
            b = i & 1
            g[b][0].wait()
            g[b][1].wait()
            if wb[b] is not None:
                wb[b].wait()

            def tokbody(j, _, i=i, b=b):
                idx = jnp.full((L,), i * L + j, jnp.int32)
                wa = plsc.load_gather(w0, [idx])
                wc = plsc.load_gather(w1, [idx])
                for cc in range(d // L):
                    yb[b][j, pl.ds(cc * L, L)] = (
                        wa * r0[b][j, pl.ds(cc * L, L)]
                        + wc * r1[b][j, pl.ds(cc * L, L)])
                return 0

            lax.fori_loop(0, L, tokbody, 0)
            if i + 2 < nchunks:
                g[b] = gathers(i + 2, b)
            wb[b] = pltpu.async_copy(yb[b], y_hbm.at[pl.ds(tb + i * L, L)],
                                     ysem[b])
        wb[0].wait()
        wb[1].wait()

    return k(outs, dest, wts)


def _ffn_body(be_ref, xs_ref, gw_ref, uw_ref, dw_ref, out_ref):
    x = xs_ref[...]
    g = jax.lax.dot_general(x, gw_ref[0], (((1,), (1,)), ((), ())),
                            preferred_element_type=jnp.float32)
    u = jax.lax.dot_general(x, uw_ref[0], (((1,), (1,)), ((), ())),
                            preferred_element_type=jnp.float32)
    h = (g * jax.nn.sigmoid(g) * u).astype(jnp.bfloat16)
    out_ref[...] = jax.lax.dot_general(h, dw_ref[0], (((1,), (1,)), ((), ())),
                                       preferred_element_type=jnp.float32)


def _grouped_ffn(xs, gate_w, up_w, down_w, block_expert):
    pt, d = xs.shape
    e, di, _ = gate_w.shape
    nb = pt // BLK
    grid_spec = pltpu.PrefetchScalarGridSpec(
        num_scalar_prefetch=1,
        grid=(nb,),
        in_specs=[
            pl.BlockSpec((BLK, d), lambda i, be: (i, 0)),
            pl.BlockSpec((1, di, d), lambda i, be: (be[i], 0, 0)),
            pl.BlockSpec((1, di, d), lambda i, be: (be[i], 0, 0)),
            pl.BlockSpec((1, d, di), lambda i, be: (be[i], 0, 0)),
        ],
        out_specs=pl.BlockSpec((BLK, d), lambda i, be: (i, 0)),
    )
    return pl.pallas_call(
        _ffn_body,
        grid_spec=grid_spec,
        out_shape=jax.ShapeDtypeStruct((pt, d), jnp.float32),
    )(block_expert, xs, gate_w, up_w, down_w)


def kernel(x, router_w, router_b, gate_w, up_w, down_w):
    b, s, d = x.shape
    t = b * s
    tk = t * K
    pt = tk + E * BLK
    nb = pt // BLK

    flat_x = x.reshape(t, d)

    experts, ranks, wts, counts = _router(flat_x, router_w, router_b)

    padded = ((counts + BLK - 1) // BLK) * BLK
    pend = jnp.cumsum(padded)
    poff = (pend - padded).astype(jnp.int32)
    block_expert = jnp.minimum(
        jnp.searchsorted(pend, jnp.arange(nb, dtype=jnp.int32) * BLK,
                         side='right'),
        E - 1).astype(jnp.int32)

    xs, dest = _dispatch(flat_x, experts, ranks, poff, pt)

    outs = _grouped_ffn(xs, gate_w, up_w, down_w, block_expert)

    y = _combine(outs, dest, wts, t, d)
    return y.reshape(b, s, d)
